# baseline (device time: 54293 ns/iter reference)
import jax
import jax.numpy as jnp
from jax import lax
from jax.experimental import pallas as pl
from jax.experimental.pallas import tpu as pltpu

N_DEV = 16
MASKS = (1, 3, 4, 8)
ORDERS = (
    (1, 3, 4, 8),
    (1, 3, 8, 4),
    (1, 4, 8, 3),
    (3, 1, 4, 8),
    (3, 1, 8, 4),
    (3, 8, 4, 1),
    (4, 1, 3, 8),
    (8, 3, 1, 4),
)
N_Q = 8
N_STEPS = 4
Q_FOR = tuple(
    {m: tuple(q for q in range(N_Q) if ORDERS[q][t] == m) for m in MASKS}
    for t in range(N_STEPS)
)


def _lsb(v: int) -> int:
    return (v & -v).bit_length() - 1


M_ORDER = [0]
for _t in (3, 2, 1, 0):
    M_ORDER.extend(r | (1 << _t) for r in list(M_ORDER))


def kernel(x):
    m, n = x.shape
    n_q = n // N_Q
    chunk_rows = m // N_DEV

    def body(
        x_ref,
        out_ref,
        recv_rs,
        rs_send_sems,
        rs_recv_sems,
        ag_send_sems,
        ag_recv_sems,
    ):
        my = lax.axis_index("i")
        coord = {
            1: (my ^ (my >> 1)) & 1,
            3: (my >> 1) & 1,
            4: (my >> 2) & 1,
            8: (my >> 3) & 1,
        }
        A = [[coord[ORDERS[q][t]] for t in range(N_STEPS)] for q in range(N_Q)]

        def cols(q):
            return pl.ds(q * n_q, n_q)

        def chunk_lo(q, r):
            return sum(
                (A[q][t] ^ ((r >> t) & 1)) * (m >> (t + 1))
                for t in range(N_STEPS)
            )

        def rs_rdma(q, t, sub, send_base, off, rows):
            return pltpu.make_async_remote_copy(
                src_ref=out_ref.at[pl.ds(send_base + off, rows), cols(q)],
                dst_ref=recv_rs.at[q, t, pl.ds(off, rows), :],
                send_sem=rs_send_sems.at[q, t, sub],
                recv_sem=rs_recv_sems.at[q, t, sub],
                device_id=(my ^ ORDERS[q][t],),
                device_id_type=pl.DeviceIdType.MESH,
            )

        def rs_issue(q, t, send_base):
            hr = m >> (t + 2)
            a_nxt = A[q][t + 1] if t < N_STEPS - 1 else 0
            pri = rs_rdma(q, t, 0, send_base, (1 - a_nxt) * hr, hr)
            pri.start()
            npri = rs_rdma(q, t, 1, send_base, a_nxt * hr, hr)
            npri.start()
            return (pri, npri)

        def ag_rdma(q, r, t):
            u = r | (1 << t)
            rr = pl.ds(chunk_lo(q, r), chunk_rows)
            return pltpu.make_async_remote_copy(
                src_ref=out_ref.at[rr, cols(q)],
                dst_ref=out_ref.at[rr, cols(q)],
                send_sem=ag_send_sems.at[q, u],
                recv_sem=ag_recv_sems.at[q, u],
                device_id=(my ^ ORDERS[q][t],),
                device_id_type=pl.DeviceIdType.MESH,
            )

        def ag_wait_rdma(q, v):
            rr = pl.ds(chunk_lo(q, v), chunk_rows)
            return pltpu.make_async_remote_copy(
                src_ref=out_ref.at[rr, cols(q)],
                dst_ref=out_ref.at[rr, cols(q)],
                send_sem=ag_send_sems.at[q, v],
                recv_sem=ag_recv_sems.at[q, v],
                device_id=(my ^ ORDERS[q][_lsb(v)],),
                device_id_type=pl.DeviceIdType.MESH,
            )

        barrier_sem = pltpu.get_barrier_semaphore()
        for mask in MASKS:
            pl.semaphore_signal(
                barrier_sem,
                inc=1,
                device_id=(my ^ mask,),
                device_id_type=pl.DeviceIdType.MESH,
            )
        pl.semaphore_wait(barrier_sem, len(MASKS))

        pending = []
        rs_inflight = [None] * N_Q
        lo = [None] * N_Q

        half0 = m // 2
        for mask in reversed(MASKS):
            for q in Q_FOR[0][mask]:
                a = A[q][0]
                send_lo = (1 - a) * half0
                rows = pl.ds(send_lo, half0)
                out_ref[rows, cols(q)] = x_ref[rows, cols(q)].astype(
                    jnp.bfloat16
                )
                rs_inflight[q] = rs_issue(q, 0, send_lo)
        for q in range(N_Q):
            a = A[q][0]
            kept_lo = a * half0
            rows = pl.ds(kept_lo, half0)
            out_ref[rows, cols(q)] = x_ref[rows, cols(q)].astype(jnp.bfloat16)
            lo[q] = kept_lo

        for t in range(N_STEPS):
            hr = m >> (t + 2)
            for mask in MASKS:
                for q in Q_FOR[t][mask]:
                    a_nxt = A[q][t + 1] if t < N_STEPS - 1 else 0
                    pri_off = (1 - a_nxt) * hr
                    pri, npri = rs_inflight[q]
                    pri.wait_recv()
                    pending.append(pri)
                    pr = pl.ds(lo[q] + pri_off, hr)
                    out_ref[pr, cols(q)] = (
                        out_ref[pr, cols(q)]
                        + recv_rs[q, t, pl.ds(pri_off, hr), :]
                    )
                    if t < N_STEPS - 1:
                        rs_inflight[q] = rs_issue(q, t + 1, lo[q] + pri_off)
                    npri.wait_recv()
                    pending.append(npri)
                    npr = pl.ds(lo[q] + a_nxt * hr, hr)
                    out_ref[npr, cols(q)] = (
                        out_ref[npr, cols(q)]
                        + recv_rs[q, t, pl.ds(a_nxt * hr, hr), :]
                    )
                    if t < N_STEPS - 1:
                        lo[q] = lo[q] + a_nxt * hr
                    else:
                        for td in reversed(range(N_STEPS)):
                            snd = ag_rdma(q, 0, td)
                            snd.start()
                            pending.append(snd)

        for v in M_ORDER[1:]:
            t_arr = _lsb(v)
            for mask in MASKS:
                for q in Q_FOR[t_arr][mask]:
                    ag_wait_rdma(q, v).wait_recv()
                    for td in reversed(range(t_arr)):
                        snd = ag_rdma(q, v, td)
                        snd.start()
                        pending.append(snd)

        for r in pending:
            r.wait_send()

    return pl.pallas_call(
        body,
        out_shape=jax.ShapeDtypeStruct((m, n), jnp.bfloat16),
        in_specs=[pl.BlockSpec(memory_space=pltpu.VMEM)],
        out_specs=pl.BlockSpec(memory_space=pltpu.VMEM),
        scratch_shapes=[
            pltpu.VMEM((N_Q, N_STEPS, m // 2, n_q), jnp.bfloat16),
            pltpu.SemaphoreType.DMA((N_Q, N_STEPS, 2)),
            pltpu.SemaphoreType.DMA((N_Q, N_STEPS, 2)),
            pltpu.SemaphoreType.DMA((N_Q, N_DEV)),
            pltpu.SemaphoreType.DMA((N_Q, N_DEV)),
        ],
        compiler_params=pltpu.CompilerParams(collective_id=0),
    )(x)


# device time: 53865 ns/iter; 1.0079x vs baseline; 1.0079x over previous
import jax
import jax.numpy as jnp
from jax import lax
from jax.experimental import pallas as pl
from jax.experimental.pallas import tpu as pltpu

N_DEV = 16
MASKS = (1, 3, 4, 8)
ORDERS = (
    (1, 3, 4, 8),
    (1, 3, 8, 4),
    (1, 4, 8, 3),
    (3, 1, 4, 8),
    (3, 1, 8, 4),
    (3, 8, 4, 1),
    (4, 1, 3, 8),
    (8, 3, 1, 4),
)
N_Q = 8
N_STEPS = 4
Q_FOR = tuple(
    {m: tuple(q for q in range(N_Q) if ORDERS[q][t] == m) for m in MASKS}
    for t in range(N_STEPS)
)


def _lsb(v: int) -> int:
    return (v & -v).bit_length() - 1


M_ORDER = [0]
for _t in (3, 2, 1, 0):
    M_ORDER.extend(r | (1 << _t) for r in list(M_ORDER))


def kernel(x):
    m, n = x.shape
    n_q = n // N_Q
    chunk_rows = m // N_DEV

    def body(
        x_ref,
        out_ref,
        recv_rs,
        rs_send_sems,
        rs_recv_sems,
        ag_send_sems,
        ag_recv_sems,
    ):
        my = lax.axis_index("i")
        coord = {
            1: (my ^ (my >> 1)) & 1,
            3: (my >> 1) & 1,
            4: (my >> 2) & 1,
            8: (my >> 3) & 1,
        }
        A = [[coord[ORDERS[q][t]] for t in range(N_STEPS)] for q in range(N_Q)]

        def cols(q):
            return pl.ds(q * n_q, n_q)

        def chunk_lo(q, r):
            return sum(
                (A[q][t] ^ ((r >> t) & 1)) * (m >> (t + 1))
                for t in range(N_STEPS)
            )

        def rs_rdma(q, t, src_lo, rows):
            return pltpu.make_async_remote_copy(
                src_ref=out_ref.at[pl.ds(src_lo, rows), cols(q)],
                dst_ref=recv_rs.at[q, t, pl.ds(0, rows), :],
                send_sem=rs_send_sems.at[q, t],
                recv_sem=rs_recv_sems.at[q, t],
                device_id=(my ^ ORDERS[q][t],),
                device_id_type=pl.DeviceIdType.MESH,
            )

        def ag_rdma(q, r, t):
            u = r | (1 << t)
            rr = pl.ds(chunk_lo(q, r), chunk_rows)
            return pltpu.make_async_remote_copy(
                src_ref=out_ref.at[rr, cols(q)],
                dst_ref=out_ref.at[rr, cols(q)],
                send_sem=ag_send_sems.at[q, u],
                recv_sem=ag_recv_sems.at[q, u],
                device_id=(my ^ ORDERS[q][t],),
                device_id_type=pl.DeviceIdType.MESH,
            )

        def ag_wait_rdma(q, v):
            rr = pl.ds(chunk_lo(q, v), chunk_rows)
            return pltpu.make_async_remote_copy(
                src_ref=out_ref.at[rr, cols(q)],
                dst_ref=out_ref.at[rr, cols(q)],
                send_sem=ag_send_sems.at[q, v],
                recv_sem=ag_recv_sems.at[q, v],
                device_id=(my ^ ORDERS[q][_lsb(v)],),
                device_id_type=pl.DeviceIdType.MESH,
            )

        barrier_sem = pltpu.get_barrier_semaphore()
        for mask in MASKS:
            pl.semaphore_signal(
                barrier_sem,
                inc=1,
                device_id=(my ^ mask,),
                device_id_type=pl.DeviceIdType.MESH,
            )
        pl.semaphore_wait(barrier_sem, len(MASKS))

        pending = []
        rs_inflight = [None] * N_Q
        lo = [None] * N_Q

        half0 = m // 2
        for mask in reversed(MASKS):
            for q in Q_FOR[0][mask]:
                a = A[q][0]
                send_lo = (1 - a) * half0
                rows = pl.ds(send_lo, half0)
                out_ref[rows, cols(q)] = x_ref[rows, cols(q)].astype(
                    jnp.bfloat16
                )
                r = rs_rdma(q, 0, send_lo, half0)
                r.start()
                rs_inflight[q] = r
        for q in range(N_Q):
            a = A[q][0]
            kept_lo = a * half0
            rows = pl.ds(kept_lo, half0)
            out_ref[rows, cols(q)] = x_ref[rows, cols(q)].astype(jnp.bfloat16)
            lo[q] = kept_lo

        for t in range(N_STEPS):
            rows_t = m >> (t + 1)
            for mask in MASKS:
                for q in Q_FOR[t][mask]:
                    r = rs_inflight[q]
                    r.wait_recv()
                    pending.append(r)
                    rr = pl.ds(lo[q], rows_t)
                    out_ref[rr, cols(q)] = (
                        out_ref[rr, cols(q)]
                        + recv_rs[q, t, pl.ds(0, rows_t), :]
                    )
                    if t < N_STEPS - 1:
                        hr = m >> (t + 2)
                        a = A[q][t + 1]
                        nxt = rs_rdma(q, t + 1, lo[q] + (1 - a) * hr, hr)
                        nxt.start()
                        rs_inflight[q] = nxt
                        lo[q] = lo[q] + a * hr
                    else:
                        for td in reversed(range(N_STEPS)):
                            snd = ag_rdma(q, 0, td)
                            snd.start()
                            pending.append(snd)

        for v in M_ORDER[1:]:
            t_arr = _lsb(v)
            for mask in MASKS:
                for q in Q_FOR[t_arr][mask]:
                    ag_wait_rdma(q, v).wait_recv()
                    for td in reversed(range(t_arr)):
                        snd = ag_rdma(q, v, td)
                        snd.start()
                        pending.append(snd)

        for r in pending:
            r.wait_send()

    return pl.pallas_call(
        body,
        out_shape=jax.ShapeDtypeStruct((m, n), jnp.bfloat16),
        in_specs=[pl.BlockSpec(memory_space=pltpu.VMEM)],
        out_specs=pl.BlockSpec(memory_space=pltpu.VMEM),
        scratch_shapes=[
            pltpu.VMEM((N_Q, N_STEPS, m // 2, n_q), jnp.bfloat16),
            pltpu.SemaphoreType.DMA((N_Q, N_STEPS)),
            pltpu.SemaphoreType.DMA((N_Q, N_STEPS)),
            pltpu.SemaphoreType.DMA((N_Q, N_DEV)),
            pltpu.SemaphoreType.DMA((N_Q, N_DEV)),
        ],
        compiler_params=pltpu.CompilerParams(collective_id=0),
    )(x)


# device time: 53181 ns/iter; 1.0209x vs baseline; 1.0129x over previous
import jax
import jax.numpy as jnp
from jax import lax
from jax.experimental import pallas as pl
from jax.experimental.pallas import tpu as pltpu

N_DEV = 16
MASKS = (1, 3, 4, 8)
ORDERS = (
    (1, 3, 4, 8),
    (1, 3, 8, 4),
    (1, 4, 8, 3),
    (3, 1, 4, 8),
    (3, 1, 8, 4),
    (3, 8, 4, 1),
    (4, 1, 3, 8),
    (8, 3, 1, 4),
)
N_Q = 8
N_STEPS = 4
Q_FOR = tuple(
    {m: tuple(q for q in range(N_Q) if ORDERS[q][t] == m) for m in MASKS}
    for t in range(N_STEPS)
)


def _lsb(v: int) -> int:
    return (v & -v).bit_length() - 1


M_ORDER = [0]
for _t in (2, 1, 0):
    M_ORDER.extend(r | (1 << _t) for r in list(M_ORDER))


def kernel(x):
    m, n = x.shape
    n_q = n // N_Q
    chunk_rows = m // (N_DEV // 2)

    def body(
        x_ref,
        out_ref,
        recv_rs,
        rs_send_sems,
        rs_recv_sems,
        ag_send_sems,
        ag_recv_sems,
    ):
        my = lax.axis_index("i")
        coord = {
            1: (my ^ (my >> 1)) & 1,
            3: (my >> 1) & 1,
            4: (my >> 2) & 1,
            8: (my >> 3) & 1,
        }
        A = [[coord[ORDERS[q][t]] for t in range(N_STEPS)] for q in range(N_Q)]

        def cols(q):
            return pl.ds(q * n_q, n_q)

        def chunk_lo(q, r):
            return sum(
                (A[q][t] ^ ((r >> t) & 1)) * (m >> (t + 1))
                for t in range(N_STEPS - 1)
            )

        def rs_rdma(q, t, src_lo, rows):
            return pltpu.make_async_remote_copy(
                src_ref=out_ref.at[pl.ds(src_lo, rows), cols(q)],
                dst_ref=recv_rs.at[q, t, pl.ds(0, rows), :],
                send_sem=rs_send_sems.at[q, t],
                recv_sem=rs_recv_sems.at[q, t],
                device_id=(my ^ ORDERS[q][t],),
                device_id_type=pl.DeviceIdType.MESH,
            )

        def ag_rdma(q, r, t):
            u = r | (1 << t)
            rr = pl.ds(chunk_lo(q, r), chunk_rows)
            return pltpu.make_async_remote_copy(
                src_ref=out_ref.at[rr, cols(q)],
                dst_ref=out_ref.at[rr, cols(q)],
                send_sem=ag_send_sems.at[q, u],
                recv_sem=ag_recv_sems.at[q, u],
                device_id=(my ^ ORDERS[q][t],),
                device_id_type=pl.DeviceIdType.MESH,
            )

        def ag_wait_rdma(q, v):
            rr = pl.ds(chunk_lo(q, v), chunk_rows)
            return pltpu.make_async_remote_copy(
                src_ref=out_ref.at[rr, cols(q)],
                dst_ref=out_ref.at[rr, cols(q)],
                send_sem=ag_send_sems.at[q, v],
                recv_sem=ag_recv_sems.at[q, v],
                device_id=(my ^ ORDERS[q][_lsb(v)],),
                device_id_type=pl.DeviceIdType.MESH,
            )

        barrier_sem = pltpu.get_barrier_semaphore()
        for mask in MASKS:
            pl.semaphore_signal(
                barrier_sem,
                inc=1,
                device_id=(my ^ mask,),
                device_id_type=pl.DeviceIdType.MESH,
            )
        pl.semaphore_wait(barrier_sem, len(MASKS))

        pending = []
        rs_inflight = [None] * N_Q
        lo = [None] * N_Q

        half0 = m // 2
        for mask in reversed(MASKS):
            for q in Q_FOR[0][mask]:
                a = A[q][0]
                send_lo = (1 - a) * half0
                rows = pl.ds(send_lo, half0)
                out_ref[rows, cols(q)] = x_ref[rows, cols(q)].astype(
                    jnp.bfloat16
                )
                r = rs_rdma(q, 0, send_lo, half0)
                r.start()
                rs_inflight[q] = r
        for q in range(N_Q):
            a = A[q][0]
            kept_lo = a * half0
            rows = pl.ds(kept_lo, half0)
            out_ref[rows, cols(q)] = x_ref[rows, cols(q)].astype(jnp.bfloat16)
            lo[q] = kept_lo

        for t in range(N_STEPS):
            rows_t = m >> (t + 1) if t < N_STEPS - 1 else chunk_rows
            for mask in MASKS:
                for q in Q_FOR[t][mask]:
                    r = rs_inflight[q]
                    r.wait_recv()
                    if t == N_STEPS - 1:
                        r.wait_send()
                    else:
                        pending.append(r)
                    rr = pl.ds(lo[q], rows_t)
                    out_ref[rr, cols(q)] = (
                        out_ref[rr, cols(q)]
                        + recv_rs[q, t, pl.ds(0, rows_t), :]
                    )
                    if t < N_STEPS - 2:
                        hr = m >> (t + 2)
                        a = A[q][t + 1]
                        nxt = rs_rdma(q, t + 1, lo[q] + (1 - a) * hr, hr)
                        nxt.start()
                        rs_inflight[q] = nxt
                        lo[q] = lo[q] + a * hr
                    elif t == N_STEPS - 2:
                        nxt = rs_rdma(q, N_STEPS - 1, lo[q], chunk_rows)
                        nxt.start()
                        rs_inflight[q] = nxt
                    else:
                        for td in reversed(range(N_STEPS - 1)):
                            snd = ag_rdma(q, 0, td)
                            snd.start()
                            pending.append(snd)

        for v in M_ORDER[1:]:
            t_arr = _lsb(v)
            for mask in MASKS:
                for q in Q_FOR[t_arr][mask]:
                    ag_wait_rdma(q, v).wait_recv()
                    for td in reversed(range(t_arr)):
                        snd = ag_rdma(q, v, td)
                        snd.start()
                        pending.append(snd)

        for r in pending:
            r.wait_send()

    return pl.pallas_call(
        body,
        out_shape=jax.ShapeDtypeStruct((m, n), jnp.bfloat16),
        in_specs=[pl.BlockSpec(memory_space=pltpu.VMEM)],
        out_specs=pl.BlockSpec(memory_space=pltpu.VMEM),
        scratch_shapes=[
            pltpu.VMEM((N_Q, N_STEPS, m // 2, n_q), jnp.bfloat16),
            pltpu.SemaphoreType.DMA((N_Q, N_STEPS)),
            pltpu.SemaphoreType.DMA((N_Q, N_STEPS)),
            pltpu.SemaphoreType.DMA((N_Q, N_DEV)),
            pltpu.SemaphoreType.DMA((N_Q, N_DEV)),
        ],
        compiler_params=pltpu.CompilerParams(collective_id=0),
    )(x)


# device time: 52776 ns/iter; 1.0287x vs baseline; 1.0077x over previous
import jax
import jax.numpy as jnp
from jax import lax
from jax.experimental import pallas as pl
from jax.experimental.pallas import tpu as pltpu

N_DEV = 16
MASKS = (1, 3, 4, 8)
ORDERS = (
    (1, 3, 4, 8),
    (1, 3, 8, 4),
    (1, 4, 8, 3),
    (3, 1, 4, 8),
    (3, 1, 8, 4),
    (3, 8, 4, 1),
    (4, 1, 3, 8),
    (8, 3, 1, 4),
)
N_Q = 8
N_STEPS = 4
Q_FOR = tuple(
    {m: tuple(q for q in range(N_Q) if ORDERS[q][t] == m) for m in MASKS}
    for t in range(N_STEPS)
)


def _lsb(v: int) -> int:
    return (v & -v).bit_length() - 1


M_ORDER = [0]
for _t in (2, 1, 0):
    M_ORDER.extend(r | (1 << _t) for r in list(M_ORDER))


def kernel(x):
    m, n = x.shape
    n_q = n // N_Q
    chunk_rows = m // (N_DEV // 2)

    def body(
        x_ref,
        out_ref,
        recv_rs,
        rs_send_sems,
        rs_recv_sems,
        ag_send_sems,
        ag_recv_sems,
    ):
        my = lax.axis_index("i")
        coord = {
            1: (my ^ (my >> 1)) & 1,
            3: (my >> 1) & 1,
            4: (my >> 2) & 1,
            8: (my >> 3) & 1,
        }
        A = [[coord[ORDERS[q][t]] for t in range(N_STEPS)] for q in range(N_Q)]

        def cols(q):
            return pl.ds(q * n_q, n_q)

        def chunk_lo(q, r):
            return sum(
                (A[q][t] ^ ((r >> t) & 1)) * (m >> (t + 1))
                for t in range(N_STEPS - 1)
            )

        def rs_rdma(q, t, src_lo, rows):
            return pltpu.make_async_remote_copy(
                src_ref=out_ref.at[pl.ds(src_lo, rows), cols(q)],
                dst_ref=recv_rs.at[q, t, pl.ds(0, rows), :],
                send_sem=rs_send_sems.at[q, t],
                recv_sem=rs_recv_sems.at[q, t],
                device_id=(my ^ ORDERS[q][t],),
                device_id_type=pl.DeviceIdType.MESH,
            )

        def ag_rdma(q, r, t):
            u = r | (1 << t)
            rr = pl.ds(chunk_lo(q, r), chunk_rows)
            return pltpu.make_async_remote_copy(
                src_ref=out_ref.at[rr, cols(q)],
                dst_ref=out_ref.at[rr, cols(q)],
                send_sem=ag_send_sems.at[q, u],
                recv_sem=ag_recv_sems.at[q, u],
                device_id=(my ^ ORDERS[q][t],),
                device_id_type=pl.DeviceIdType.MESH,
            )

        def ag_wait_rdma(q, v):
            rr = pl.ds(chunk_lo(q, v), chunk_rows)
            return pltpu.make_async_remote_copy(
                src_ref=out_ref.at[rr, cols(q)],
                dst_ref=out_ref.at[rr, cols(q)],
                send_sem=ag_send_sems.at[q, v],
                recv_sem=ag_recv_sems.at[q, v],
                device_id=(my ^ ORDERS[q][_lsb(v)],),
                device_id_type=pl.DeviceIdType.MESH,
            )

        barrier_sem = pltpu.get_barrier_semaphore()
        for mask in MASKS:
            pl.semaphore_signal(
                barrier_sem,
                inc=1,
                device_id=(my ^ mask,),
                device_id_type=pl.DeviceIdType.MESH,
            )

        half0 = m // 2
        for mask in reversed(MASKS):
            for q in Q_FOR[0][mask]:
                a = A[q][0]
                send_lo = (1 - a) * half0
                rows = pl.ds(send_lo, half0)
                out_ref[rows, cols(q)] = x_ref[rows, cols(q)].astype(
                    jnp.bfloat16
                )

        pl.semaphore_wait(barrier_sem, len(MASKS))

        pending = []
        rs_inflight = [None] * N_Q
        lo = [None] * N_Q

        for mask in reversed(MASKS):
            for q in Q_FOR[0][mask]:
                a = A[q][0]
                send_lo = (1 - a) * half0
                r = rs_rdma(q, 0, send_lo, half0)
                r.start()
                rs_inflight[q] = r
        for q in range(N_Q):
            a = A[q][0]
            kept_lo = a * half0
            rows = pl.ds(kept_lo, half0)
            out_ref[rows, cols(q)] = x_ref[rows, cols(q)].astype(jnp.bfloat16)
            lo[q] = kept_lo

        for t in range(N_STEPS):
            rows_t = m >> (t + 1) if t < N_STEPS - 1 else chunk_rows
            for mask in MASKS:
                for q in Q_FOR[t][mask]:
                    r = rs_inflight[q]
                    r.wait_recv()
                    if t == N_STEPS - 1:
                        r.wait_send()
                    else:
                        pending.append(r)
                    rr = pl.ds(lo[q], rows_t)
                    out_ref[rr, cols(q)] = (
                        out_ref[rr, cols(q)]
                        + recv_rs[q, t, pl.ds(0, rows_t), :]
                    )
                    if t < N_STEPS - 2:
                        hr = m >> (t + 2)
                        a = A[q][t + 1]
                        nxt = rs_rdma(q, t + 1, lo[q] + (1 - a) * hr, hr)
                        nxt.start()
                        rs_inflight[q] = nxt
                        lo[q] = lo[q] + a * hr
                    elif t == N_STEPS - 2:
                        nxt = rs_rdma(q, N_STEPS - 1, lo[q], chunk_rows)
                        nxt.start()
                        rs_inflight[q] = nxt
                    else:
                        for td in reversed(range(N_STEPS - 1)):
                            snd = ag_rdma(q, 0, td)
                            snd.start()
                            pending.append(snd)

        for v in M_ORDER[1:]:
            t_arr = _lsb(v)
            for mask in MASKS:
                for q in Q_FOR[t_arr][mask]:
                    ag_wait_rdma(q, v).wait_recv()
                    for td in reversed(range(t_arr)):
                        snd = ag_rdma(q, v, td)
                        snd.start()
                        pending.append(snd)

        for r in pending:
            r.wait_send()

    return pl.pallas_call(
        body,
        out_shape=jax.ShapeDtypeStruct((m, n), jnp.bfloat16),
        in_specs=[pl.BlockSpec(memory_space=pltpu.VMEM)],
        out_specs=pl.BlockSpec(memory_space=pltpu.VMEM),
        scratch_shapes=[
            pltpu.VMEM((N_Q, N_STEPS, m // 2, n_q), jnp.bfloat16),
            pltpu.SemaphoreType.DMA((N_Q, N_STEPS)),
            pltpu.SemaphoreType.DMA((N_Q, N_STEPS)),
            pltpu.SemaphoreType.DMA((N_Q, N_DEV)),
            pltpu.SemaphoreType.DMA((N_Q, N_DEV)),
        ],
        compiler_params=pltpu.CompilerParams(collective_id=0),
    )(x)
